# baseline (speedup 1.0000x reference)
import jax
import jax.numpy as jnp
from jax import lax
from jax.experimental import pallas as pl
from jax.experimental.pallas import tpu as pltpu
from jax.experimental.pallas import tpu_sc as plsc

N_NODES = 10000
ROW = 10000
BATCH = 8192
NC, NS = 2, 16
NW = NC * NS
PER_W = BATCH // NW
K = 4
NG = PER_W // K
LANES = 16
CHUNKS = ROW // LANES
U = 5
NJ = CHUNKS // U


def _pair_dot(r1_v, r2_v, p):
    def body(j, accs):
        base = j * (U * LANES)
        out = []
        for u in range(U):
            x = r1_v[p, pl.ds(base + u * LANES, LANES)]
            y = r2_v[p, pl.ds(base + u * LANES, LANES)]
            out.append(accs[u] + x * y)
        return tuple(out)

    accs = tuple(jnp.zeros((LANES,), jnp.float32) for _ in range(U))
    accs = lax.fori_loop(0, NJ, body, accs)
    tot = accs[0]
    for u in range(1, U):
        tot = tot + accs[u]
    lane = lax.broadcasted_iota(jnp.int32, (LANES,), 0)
    for sh in (8, 4, 2, 1):
        idx = jnp.bitwise_and(lane + sh, LANES - 1)
        tot = tot + tot.at[idx].get(mode="promise_in_bounds",
                                    unique_indices=True)
    return tot


QPS = LANES // K
NSG = PER_W // LANES


def _sc_body(a_hbm, n1_hbm, n2_hbm, out_hbm,
             idx1_v, idx2_v, r1_v, r2_v, out_v, sem1, sem2):
    wid = lax.axis_index("s") * NC + lax.axis_index("c")
    base = wid * PER_W
    pltpu.sync_copy(n1_hbm.at[pl.ds(wid * NG, NG)], idx1_v)
    pltpu.sync_copy(n2_hbm.at[pl.ds(wid * NG, NG)], idx2_v)
    lane = lax.broadcasted_iota(jnp.int32, (LANES,), 0)

    def supergroup(sg, carry):
        vec = jnp.zeros((LANES,), jnp.float32)
        off = sg * LANES
        for q in range(QPS):
            g = sg * QPS + q
            c1 = pltpu.async_copy(a_hbm.at[idx1_v.at[g]], r1_v, sem1)
            c2 = pltpu.async_copy(a_hbm.at[idx2_v.at[g]], r2_v, sem2)
            c1.wait()
            c2.wait()
            for p in range(K):
                s = _pair_dot(r1_v, r2_v, p)
                vec = jnp.where(lane == (q * K + p), s, vec)
        out_v[pl.ds(off, LANES)] = vec
        return carry

    lax.fori_loop(0, NSG, supergroup, 0)
    pltpu.sync_copy(out_v, out_hbm.at[pl.ds(base, PER_W)])


def kernel(A, nodes1, nodes2):
    n1 = nodes1.astype(jnp.int32).reshape(BATCH // K, K)
    n2 = nodes2.astype(jnp.int32).reshape(BATCH // K, K)
    mesh = plsc.VectorSubcoreMesh(core_axis_name="c", subcore_axis_name="s")
    f = pl.kernel(
        _sc_body,
        out_type=jax.ShapeDtypeStruct((BATCH,), jnp.float32),
        mesh=mesh,
        compiler_params=pltpu.CompilerParams(use_tc_tiling_on_sc=False),
        scratch_types=[
            pltpu.VMEM((NG, K), jnp.int32),
            pltpu.VMEM((NG, K), jnp.int32),
            pltpu.VMEM((K, ROW), jnp.float32),
            pltpu.VMEM((K, ROW), jnp.float32),
            pltpu.VMEM((PER_W,), jnp.float32),
            pltpu.SemaphoreType.DMA,
            pltpu.SemaphoreType.DMA,
        ],
    )
    return f(A, n1, n2)


# speedup vs baseline: 1.2202x; 1.2202x over previous
import jax
import jax.numpy as jnp
from jax import lax
from jax.experimental import pallas as pl
from jax.experimental.pallas import tpu as pltpu
from jax.experimental.pallas import tpu_sc as plsc

ROW = 10000
BATCH = 8192
NC, NS = 2, 16
NW = NC * NS
PER_W = BATCH // NW
K = 2
NGR = PER_W // K
LANES = 16
CHUNKS = ROW // LANES
U = 25
NJ = CHUNKS // U
NACC = 5
GPS = LANES // K
NSG = PER_W // LANES


def _pair_dot(r1_v, r2_v, p):
    def body(j, accs):
        base = j * (U * LANES)
        accs = list(accs)
        for u in range(U):
            x = r1_v[p, pl.ds(base + u * LANES, LANES)]
            y = r2_v[p, pl.ds(base + u * LANES, LANES)]
            accs[u % NACC] = accs[u % NACC] + x * y
        return tuple(accs)

    accs = tuple(jnp.zeros((LANES,), jnp.float32) for _ in range(NACC))
    accs = lax.fori_loop(0, NJ, body, accs)
    tot = accs[0]
    for u in range(1, NACC):
        tot = tot + accs[u]
    lane = lax.broadcasted_iota(jnp.int32, (LANES,), 0)
    for sh in (8, 4, 2, 1):
        idx = jnp.bitwise_and(lane + sh, LANES - 1)
        tot = tot + tot.at[idx].get(mode="promise_in_bounds",
                                    unique_indices=True)
    return tot


def _sc_body(a_hbm, n1_hbm, n2_hbm, out_hbm,
             idx1_v, idx2_v, r1a_v, r1b_v, r2a_v, r2b_v, out_v,
             s1a, s1b, s2a, s2b):
    wid = lax.axis_index("s") * NC + lax.axis_index("c")
    pltpu.sync_copy(n1_hbm.at[pl.ds(wid * NGR, NGR)], idx1_v)
    pltpu.sync_copy(n2_hbm.at[pl.ds(wid * NGR, NGR)], idx2_v)
    lane = lax.broadcasted_iota(jnp.int32, (LANES,), 0)

    r1 = (r1a_v, r1b_v)
    r2 = (r2a_v, r2b_v)
    sem1 = (s1a, s1b)
    sem2 = (s2a, s2b)

    def fire(g, par):
        pltpu.async_copy(a_hbm.at[idx1_v.at[g]], r1[par], sem1[par])
        pltpu.async_copy(a_hbm.at[idx2_v.at[g]], r2[par], sem2[par])

    def drain(par):
        pltpu.make_async_copy(a_hbm.at[idx1_v.at[0]], r1[par], sem1[par]).wait()
        pltpu.make_async_copy(a_hbm.at[idx2_v.at[0]], r2[par], sem2[par]).wait()

    fire(0, 0)

    def supergroup(sg, carry):
        vec = jnp.zeros((LANES,), jnp.float32)
        for q in range(GPS):
            par = q % 2
            g = sg * GPS + q
            drain(par)
            g_next = g + 1

            @pl.when(g_next < NGR)
            def _():
                fire(g_next, (q + 1) % 2)

            for p in range(K):
                s = _pair_dot(r1[par], r2[par], p)
                vec = jnp.where(lane == (q * K + p), s, vec)
        out_v[pl.ds(sg * LANES, LANES)] = vec
        return carry

    lax.fori_loop(0, NSG, supergroup, 0)
    pltpu.sync_copy(out_v, out_hbm.at[pl.ds(wid * PER_W, PER_W)])


def kernel(A, nodes1, nodes2):
    n1 = nodes1.astype(jnp.int32).reshape(BATCH // K, K)
    n2 = nodes2.astype(jnp.int32).reshape(BATCH // K, K)
    mesh = plsc.VectorSubcoreMesh(core_axis_name="c", subcore_axis_name="s")
    f = pl.kernel(
        _sc_body,
        out_type=jax.ShapeDtypeStruct((BATCH,), jnp.float32),
        mesh=mesh,
        compiler_params=pltpu.CompilerParams(use_tc_tiling_on_sc=False),
        scratch_types=[
            pltpu.VMEM((NGR, K), jnp.int32),
            pltpu.VMEM((NGR, K), jnp.int32),
            pltpu.VMEM((K, ROW), jnp.float32),
            pltpu.VMEM((K, ROW), jnp.float32),
            pltpu.VMEM((K, ROW), jnp.float32),
            pltpu.VMEM((K, ROW), jnp.float32),
            pltpu.VMEM((PER_W,), jnp.float32),
            pltpu.SemaphoreType.DMA,
            pltpu.SemaphoreType.DMA,
            pltpu.SemaphoreType.DMA,
            pltpu.SemaphoreType.DMA,
        ],
    )
    return f(A, n1, n2)


# speedup vs baseline: 2.4838x; 2.0355x over previous
import jax
import jax.numpy as jnp
from jax import lax
from jax.experimental import pallas as pl
from jax.experimental.pallas import tpu as pltpu
from jax.experimental.pallas import tpu_sc as plsc

ROW = 10000
MAIN = 9984
TAIL = ROW - MAIN
TPAD = 128
BATCH = 8192
NC, NS = 2, 16
NW = NC * NS
PER_W = BATCH // NW
K = 2
NGR = PER_W // K
LANES = 16
CHUNKS = MAIN // LANES
U = 24
NJ = CHUNKS // U
NACC = 6
TCHUNKS = TPAD // LANES
GPS = LANES // K
NSG = PER_W // LANES


def _pair_dot(rm1, rt1, rm2, rt2, p):
    def body(j, accs):
        base = j * (U * LANES)
        accs = list(accs)
        for u in range(U):
            x = rm1[p, pl.ds(base + u * LANES, LANES)]
            y = rm2[p, pl.ds(base + u * LANES, LANES)]
            accs[u % NACC] = accs[u % NACC] + x * y
        return tuple(accs)

    accs = tuple(jnp.zeros((LANES,), jnp.float32) for _ in range(NACC))
    accs = lax.fori_loop(0, NJ, body, accs)
    accs = list(accs)
    for t in range(TCHUNKS):
        x = rt1[p, pl.ds(t * LANES, LANES)]
        y = rt2[p, pl.ds(t * LANES, LANES)]
        accs[t % NACC] = accs[t % NACC] + x * y
    tot = accs[0]
    for u in range(1, NACC):
        tot = tot + accs[u]
    lane = lax.broadcasted_iota(jnp.int32, (LANES,), 0)
    for sh in (8, 4, 2, 1):
        idx = jnp.bitwise_and(lane + sh, LANES - 1)
        tot = tot + tot.at[idx].get(mode="promise_in_bounds",
                                    unique_indices=True)
    return tot


def _sc_body(a_hbm, atail_hbm, n1_hbm, n2_hbm, out_hbm,
             idx1_v, idx2_v,
             m1a, m1b, m2a, m2b, t1a, t1b, t2a, t2b, out_v,
             s1a, s1b, s2a, s2b):
    wid = lax.axis_index("s") * NC + lax.axis_index("c")
    pltpu.sync_copy(n1_hbm.at[pl.ds(wid * NGR, NGR)], idx1_v)
    pltpu.sync_copy(n2_hbm.at[pl.ds(wid * NGR, NGR)], idx2_v)
    lane = lax.broadcasted_iota(jnp.int32, (LANES,), 0)

    a_main = a_hbm.at[:, pl.ds(0, MAIN)]
    m1 = (m1a, m1b)
    m2 = (m2a, m2b)
    t1 = (t1a, t1b)
    t2 = (t2a, t2b)
    sem1 = (s1a, s1b)
    sem2 = (s2a, s2b)

    def fire(g, par):
        pltpu.async_copy(a_main.at[idx1_v.at[g]], m1[par], sem1[par])
        pltpu.async_copy(atail_hbm.at[idx1_v.at[g]], t1[par], sem1[par])
        pltpu.async_copy(a_main.at[idx2_v.at[g]], m2[par], sem2[par])
        pltpu.async_copy(atail_hbm.at[idx2_v.at[g]], t2[par], sem2[par])

    def drain(par):
        pltpu.make_async_copy(a_main.at[idx1_v.at[0]], m1[par], sem1[par]).wait()
        pltpu.make_async_copy(atail_hbm.at[idx1_v.at[0]], t1[par], sem1[par]).wait()
        pltpu.make_async_copy(a_main.at[idx2_v.at[0]], m2[par], sem2[par]).wait()
        pltpu.make_async_copy(atail_hbm.at[idx2_v.at[0]], t2[par], sem2[par]).wait()

    fire(0, 0)

    def supergroup(sg, carry):
        vec = jnp.zeros((LANES,), jnp.float32)
        for q in range(GPS):
            par = q % 2
            g = sg * GPS + q
            drain(par)
            g_next = g + 1

            @pl.when(g_next < NGR)
            def _():
                fire(g_next, (q + 1) % 2)

            for p in range(K):
                s = _pair_dot(m1[par], t1[par], m2[par], t2[par], p)
                vec = jnp.where(lane == (q * K + p), s, vec)
        out_v[pl.ds(sg * LANES, LANES)] = vec
        return carry

    lax.fori_loop(0, NSG, supergroup, 0)
    pltpu.sync_copy(out_v, out_hbm.at[pl.ds(wid * PER_W, PER_W)])


def kernel(A, nodes1, nodes2):
    n1 = nodes1.astype(jnp.int32).reshape(BATCH // K, K)
    n2 = nodes2.astype(jnp.int32).reshape(BATCH // K, K)
    a_tail = jnp.pad(A[:, MAIN:], ((0, 0), (0, TPAD - TAIL)))
    mesh = plsc.VectorSubcoreMesh(core_axis_name="c", subcore_axis_name="s")
    f = pl.kernel(
        _sc_body,
        out_type=jax.ShapeDtypeStruct((BATCH,), jnp.float32),
        mesh=mesh,
        compiler_params=pltpu.CompilerParams(use_tc_tiling_on_sc=True),
        scratch_types=[
            pltpu.VMEM((NGR, K), jnp.int32),
            pltpu.VMEM((NGR, K), jnp.int32),
            pltpu.VMEM((K, MAIN), jnp.float32),
            pltpu.VMEM((K, MAIN), jnp.float32),
            pltpu.VMEM((K, MAIN), jnp.float32),
            pltpu.VMEM((K, MAIN), jnp.float32),
            pltpu.VMEM((K, TPAD), jnp.float32),
            pltpu.VMEM((K, TPAD), jnp.float32),
            pltpu.VMEM((K, TPAD), jnp.float32),
            pltpu.VMEM((K, TPAD), jnp.float32),
            pltpu.VMEM((PER_W,), jnp.float32),
            pltpu.SemaphoreType.DMA,
            pltpu.SemaphoreType.DMA,
            pltpu.SemaphoreType.DMA,
            pltpu.SemaphoreType.DMA,
        ],
    )
    return f(A, a_tail, n1, n2)


# speedup vs baseline: 2.9100x; 1.1716x over previous
import jax
import jax.numpy as jnp
from jax import lax
from jax.experimental import pallas as pl
from jax.experimental.pallas import tpu as pltpu
from jax.experimental.pallas import tpu_sc as plsc

ROW = 10000
MAIN = 9984
TAIL = ROW - MAIN
TPAD = 128
BATCH = 8192
SC_N = 6144
TC_N = BATCH - SC_N
NC, NS = 2, 16
NW = NC * NS
PER_W = SC_N // NW
K = 2
NGR = PER_W // K
LANES = 16
CHUNKS = MAIN // LANES
U = 24
NJ = CHUNKS // U
NACC = 6
TCHUNKS = TPAD // LANES
GPS = LANES // K
NSG = PER_W // LANES

BP = 128
NB = TC_N // BP


def _pair_dot(rm1, rt1, rm2, rt2, p):
    def body(j, accs):
        base = j * (U * LANES)
        accs = list(accs)
        for u in range(U):
            x = rm1[p, pl.ds(base + u * LANES, LANES)]
            y = rm2[p, pl.ds(base + u * LANES, LANES)]
            accs[u % NACC] = accs[u % NACC] + x * y
        return tuple(accs)

    accs = tuple(jnp.zeros((LANES,), jnp.float32) for _ in range(NACC))
    accs = lax.fori_loop(0, NJ, body, accs)
    accs = list(accs)
    for t in range(TCHUNKS):
        x = rt1[p, pl.ds(t * LANES, LANES)]
        y = rt2[p, pl.ds(t * LANES, LANES)]
        accs[t % NACC] = accs[t % NACC] + x * y
    tot = accs[0]
    for u in range(1, NACC):
        tot = tot + accs[u]
    lane = lax.broadcasted_iota(jnp.int32, (LANES,), 0)
    for sh in (8, 4, 2, 1):
        idx = jnp.bitwise_and(lane + sh, LANES - 1)
        tot = tot + tot.at[idx].get(mode="promise_in_bounds",
                                    unique_indices=True)
    return tot


def _sc_body(a_hbm, atail_hbm, n1_hbm, n2_hbm, out_hbm,
             idx1_v, idx2_v,
             m1a, m1b, m2a, m2b, t1a, t1b, t2a, t2b, out_v,
             s1a, s1b, s2a, s2b):
    wid = lax.axis_index("s") * NC + lax.axis_index("c")
    pltpu.sync_copy(n1_hbm.at[pl.ds(wid * NGR, NGR)], idx1_v)
    pltpu.sync_copy(n2_hbm.at[pl.ds(wid * NGR, NGR)], idx2_v)
    lane = lax.broadcasted_iota(jnp.int32, (LANES,), 0)

    a_main = a_hbm.at[:, pl.ds(0, MAIN)]
    m1 = (m1a, m1b)
    m2 = (m2a, m2b)
    t1 = (t1a, t1b)
    t2 = (t2a, t2b)
    sem1 = (s1a, s1b)
    sem2 = (s2a, s2b)

    def fire(g, par):
        pltpu.async_copy(a_main.at[idx1_v.at[g]], m1[par], sem1[par])
        pltpu.async_copy(atail_hbm.at[idx1_v.at[g]], t1[par], sem1[par])
        pltpu.async_copy(a_main.at[idx2_v.at[g]], m2[par], sem2[par])
        pltpu.async_copy(atail_hbm.at[idx2_v.at[g]], t2[par], sem2[par])

    def drain(par):
        pltpu.make_async_copy(a_main.at[idx1_v.at[0]], m1[par], sem1[par]).wait()
        pltpu.make_async_copy(atail_hbm.at[idx1_v.at[0]], t1[par], sem1[par]).wait()
        pltpu.make_async_copy(a_main.at[idx2_v.at[0]], m2[par], sem2[par]).wait()
        pltpu.make_async_copy(atail_hbm.at[idx2_v.at[0]], t2[par], sem2[par]).wait()

    fire(0, 0)

    def supergroup(sg, carry):
        vec = jnp.zeros((LANES,), jnp.float32)
        for q in range(GPS):
            par = q % 2
            g = sg * GPS + q
            drain(par)
            g_next = g + 1

            @pl.when(g_next < NGR)
            def _():
                fire(g_next, (q + 1) % 2)

            for p in range(K):
                s = _pair_dot(m1[par], t1[par], m2[par], t2[par], p)
                vec = jnp.where(lane == (q * K + p), s, vec)
        out_v[pl.ds(sg * LANES, LANES)] = vec
        return carry

    lax.fori_loop(0, NSG, supergroup, 0)
    pltpu.sync_copy(out_v, out_hbm.at[pl.ds(wid * PER_W, PER_W)])


def _tc_body(n1_sref, n2_sref, a_ref, out_ref, rows1, rows2, sems):
    i = pl.program_id(0)

    def fire(step, par):
        for p in range(BP):
            i1 = n1_sref[step * BP + p]
            i2 = n2_sref[step * BP + p]
            pltpu.make_async_copy(
                a_ref.at[pl.ds(i1, 1)], rows1.at[par, pl.ds(p, 1)],
                sems.at[par]).start()
            pltpu.make_async_copy(
                a_ref.at[pl.ds(i2, 1)], rows2.at[par, pl.ds(p, 1)],
                sems.at[par]).start()

    def drain(step, par):
        for p in range(BP):
            i1 = n1_sref[step * BP + p]
            i2 = n2_sref[step * BP + p]
            pltpu.make_async_copy(
                a_ref.at[pl.ds(i1, 1)], rows1.at[par, pl.ds(p, 1)],
                sems.at[par]).wait()
            pltpu.make_async_copy(
                a_ref.at[pl.ds(i2, 1)], rows2.at[par, pl.ds(p, 1)],
                sems.at[par]).wait()

    @pl.when(i == 0)
    def _():
        fire(0, 0)

    par = lax.rem(i, 2)

    @pl.when(i + 1 < NB)
    def _():
        fire(i + 1, lax.rem(i + 1, 2))

    drain(i, par)
    r1 = rows1[par]
    r2 = rows2[par]
    out_ref[0, 0, :] = jnp.sum(r1 * r2, axis=1)


def kernel(A, nodes1, nodes2):
    n1 = nodes1.astype(jnp.int32)
    n2 = nodes2.astype(jnp.int32)
    n1_sc = n1[:SC_N].reshape(SC_N // K, K)
    n2_sc = n2[:SC_N].reshape(SC_N // K, K)
    a_tail = jnp.pad(A[:, MAIN:], ((0, 0), (0, TPAD - TAIL)))
    mesh = plsc.VectorSubcoreMesh(core_axis_name="c", subcore_axis_name="s")
    sc_fn = pl.kernel(
        _sc_body,
        out_type=jax.ShapeDtypeStruct((SC_N,), jnp.float32),
        mesh=mesh,
        compiler_params=pltpu.CompilerParams(use_tc_tiling_on_sc=True),
        scratch_types=[
            pltpu.VMEM((NGR, K), jnp.int32),
            pltpu.VMEM((NGR, K), jnp.int32),
            pltpu.VMEM((K, MAIN), jnp.float32),
            pltpu.VMEM((K, MAIN), jnp.float32),
            pltpu.VMEM((K, MAIN), jnp.float32),
            pltpu.VMEM((K, MAIN), jnp.float32),
            pltpu.VMEM((K, TPAD), jnp.float32),
            pltpu.VMEM((K, TPAD), jnp.float32),
            pltpu.VMEM((K, TPAD), jnp.float32),
            pltpu.VMEM((K, TPAD), jnp.float32),
            pltpu.VMEM((PER_W,), jnp.float32),
            pltpu.SemaphoreType.DMA,
            pltpu.SemaphoreType.DMA,
            pltpu.SemaphoreType.DMA,
            pltpu.SemaphoreType.DMA,
        ],
    )
    sc_out = sc_fn(A, a_tail, n1_sc, n2_sc)

    tc_fn = pl.pallas_call(
        _tc_body,
        grid_spec=pltpu.PrefetchScalarGridSpec(
            num_scalar_prefetch=2,
            grid=(NB,),
            in_specs=[pl.BlockSpec(memory_space=pl.ANY)],
            out_specs=pl.BlockSpec((1, 1, BP), lambda i, n1, n2: (i, 0, 0)),
            scratch_shapes=[
                pltpu.VMEM((2, BP, ROW), jnp.float32),
                pltpu.VMEM((2, BP, ROW), jnp.float32),
                pltpu.SemaphoreType.DMA((2,)),
            ],
        ),
        out_shape=jax.ShapeDtypeStruct((NB, 1, BP), jnp.float32),
    )
    tc_out = tc_fn(n1[SC_N:], n2[SC_N:], A).reshape(TC_N)
    return jnp.concatenate([sc_out, tc_out])


# speedup vs baseline: 3.3241x; 1.1423x over previous
import jax
import jax.numpy as jnp
from jax import lax
from jax.experimental import pallas as pl
from jax.experimental.pallas import tpu as pltpu
from jax.experimental.pallas import tpu_sc as plsc

ROW = 10000
MAIN = 9984
TAIL = ROW - MAIN
TPAD = 128
BATCH = 8192
SC_N = 4608
TC_N = BATCH - SC_N
NC, NS = 2, 16
NW = NC * NS
PER_W = SC_N // NW
K = 2
NGR = PER_W // K
LANES = 16
CHUNKS = MAIN // LANES
U = 24
NJ = CHUNKS // U
NACC = 6
TCHUNKS = TPAD // LANES
GPS = LANES // K
NSG = PER_W // LANES

BP = 128
NB = TC_N // BP


def _pair_dot(rm1, rt1, rm2, rt2, p):
    def body(j, accs):
        base = j * (U * LANES)
        accs = list(accs)
        for u in range(U):
            x = rm1[p, pl.ds(base + u * LANES, LANES)]
            y = rm2[p, pl.ds(base + u * LANES, LANES)]
            accs[u % NACC] = accs[u % NACC] + x * y
        return tuple(accs)

    accs = tuple(jnp.zeros((LANES,), jnp.float32) for _ in range(NACC))
    accs = lax.fori_loop(0, NJ, body, accs)
    accs = list(accs)
    for t in range(TCHUNKS):
        x = rt1[p, pl.ds(t * LANES, LANES)]
        y = rt2[p, pl.ds(t * LANES, LANES)]
        accs[t % NACC] = accs[t % NACC] + x * y
    tot = accs[0]
    for u in range(1, NACC):
        tot = tot + accs[u]
    lane = lax.broadcasted_iota(jnp.int32, (LANES,), 0)
    for sh in (8, 4, 2, 1):
        idx = jnp.bitwise_and(lane + sh, LANES - 1)
        tot = tot + tot.at[idx].get(mode="promise_in_bounds",
                                    unique_indices=True)
    return tot


def _sc_body(a_hbm, atail_hbm, n1_hbm, n2_hbm, out_hbm,
             idx1_v, idx2_v,
             m1a, m1b, m2a, m2b, t1a, t1b, t2a, t2b, out_v,
             s1a, s1b, s2a, s2b):
    wid = lax.axis_index("s") * NC + lax.axis_index("c")
    pltpu.sync_copy(n1_hbm.at[pl.ds(wid * NGR, NGR)], idx1_v)
    pltpu.sync_copy(n2_hbm.at[pl.ds(wid * NGR, NGR)], idx2_v)
    lane = lax.broadcasted_iota(jnp.int32, (LANES,), 0)

    a_main = a_hbm.at[:, pl.ds(0, MAIN)]
    m1 = (m1a, m1b)
    m2 = (m2a, m2b)
    t1 = (t1a, t1b)
    t2 = (t2a, t2b)
    sem1 = (s1a, s1b)
    sem2 = (s2a, s2b)

    def fire(g, par):
        pltpu.async_copy(a_main.at[idx1_v.at[g]], m1[par], sem1[par])
        pltpu.async_copy(atail_hbm.at[idx1_v.at[g]], t1[par], sem1[par])
        pltpu.async_copy(a_main.at[idx2_v.at[g]], m2[par], sem2[par])
        pltpu.async_copy(atail_hbm.at[idx2_v.at[g]], t2[par], sem2[par])

    def drain(par):
        pltpu.make_async_copy(a_main.at[idx1_v.at[0]], m1[par], sem1[par]).wait()
        pltpu.make_async_copy(atail_hbm.at[idx1_v.at[0]], t1[par], sem1[par]).wait()
        pltpu.make_async_copy(a_main.at[idx2_v.at[0]], m2[par], sem2[par]).wait()
        pltpu.make_async_copy(atail_hbm.at[idx2_v.at[0]], t2[par], sem2[par]).wait()

    fire(0, 0)

    def supergroup(sg, carry):
        vec = jnp.zeros((LANES,), jnp.float32)
        for q in range(GPS):
            par = q % 2
            g = sg * GPS + q
            drain(par)
            g_next = g + 1

            @pl.when(g_next < NGR)
            def _():
                fire(g_next, (q + 1) % 2)

            for p in range(K):
                s = _pair_dot(m1[par], t1[par], m2[par], t2[par], p)
                vec = jnp.where(lane == (q * K + p), s, vec)
        out_v[pl.ds(sg * LANES, LANES)] = vec
        return carry

    lax.fori_loop(0, NSG, supergroup, 0)
    pltpu.sync_copy(out_v, out_hbm.at[pl.ds(wid * PER_W, PER_W)])


def _tc_body(n1_sref, n2_sref, a_ref, out_ref, rows1, rows2, sems):
    i = pl.program_id(0)

    def fire(step, par):
        for p in range(BP):
            i1 = n1_sref[step * BP + p]
            i2 = n2_sref[step * BP + p]
            pltpu.make_async_copy(
                a_ref.at[pl.ds(i1, 1)], rows1.at[par, pl.ds(p, 1)],
                sems.at[par]).start()
            pltpu.make_async_copy(
                a_ref.at[pl.ds(i2, 1)], rows2.at[par, pl.ds(p, 1)],
                sems.at[par]).start()

    def drain(step, par):
        for p in range(BP):
            i1 = n1_sref[step * BP + p]
            i2 = n2_sref[step * BP + p]
            pltpu.make_async_copy(
                a_ref.at[pl.ds(i1, 1)], rows1.at[par, pl.ds(p, 1)],
                sems.at[par]).wait()
            pltpu.make_async_copy(
                a_ref.at[pl.ds(i2, 1)], rows2.at[par, pl.ds(p, 1)],
                sems.at[par]).wait()

    @pl.when(i == 0)
    def _():
        fire(0, 0)

    par = lax.rem(i, 2)

    @pl.when(i + 1 < NB)
    def _():
        fire(i + 1, lax.rem(i + 1, 2))

    drain(i, par)
    r1 = rows1[par]
    r2 = rows2[par]
    out_ref[0, 0, :] = jnp.sum(r1 * r2, axis=1)


def kernel(A, nodes1, nodes2):
    n1 = nodes1.astype(jnp.int32)
    n2 = nodes2.astype(jnp.int32)
    n1_sc = n1[:SC_N].reshape(SC_N // K, K)
    n2_sc = n2[:SC_N].reshape(SC_N // K, K)
    a_tail = jnp.pad(A[:, MAIN:], ((0, 0), (0, TPAD - TAIL)))
    mesh = plsc.VectorSubcoreMesh(core_axis_name="c", subcore_axis_name="s")
    sc_fn = pl.kernel(
        _sc_body,
        out_type=jax.ShapeDtypeStruct((SC_N,), jnp.float32),
        mesh=mesh,
        compiler_params=pltpu.CompilerParams(use_tc_tiling_on_sc=True),
        scratch_types=[
            pltpu.VMEM((NGR, K), jnp.int32),
            pltpu.VMEM((NGR, K), jnp.int32),
            pltpu.VMEM((K, MAIN), jnp.float32),
            pltpu.VMEM((K, MAIN), jnp.float32),
            pltpu.VMEM((K, MAIN), jnp.float32),
            pltpu.VMEM((K, MAIN), jnp.float32),
            pltpu.VMEM((K, TPAD), jnp.float32),
            pltpu.VMEM((K, TPAD), jnp.float32),
            pltpu.VMEM((K, TPAD), jnp.float32),
            pltpu.VMEM((K, TPAD), jnp.float32),
            pltpu.VMEM((PER_W,), jnp.float32),
            pltpu.SemaphoreType.DMA,
            pltpu.SemaphoreType.DMA,
            pltpu.SemaphoreType.DMA,
            pltpu.SemaphoreType.DMA,
        ],
    )
    sc_out = sc_fn(A, a_tail, n1_sc, n2_sc)

    tc_fn = pl.pallas_call(
        _tc_body,
        grid_spec=pltpu.PrefetchScalarGridSpec(
            num_scalar_prefetch=2,
            grid=(NB,),
            in_specs=[pl.BlockSpec(memory_space=pl.ANY)],
            out_specs=pl.BlockSpec((1, 1, BP), lambda i, n1, n2: (i, 0, 0)),
            scratch_shapes=[
                pltpu.VMEM((2, BP, ROW), jnp.float32),
                pltpu.VMEM((2, BP, ROW), jnp.float32),
                pltpu.SemaphoreType.DMA((2,)),
            ],
        ),
        out_shape=jax.ShapeDtypeStruct((NB, 1, BP), jnp.float32),
    )
    tc_out = tc_fn(n1[SC_N:], n2[SC_N:], A).reshape(TC_N)
    return jnp.concatenate([sc_out, tc_out])


# speedup vs baseline: 3.3273x; 1.0010x over previous
import jax
import jax.numpy as jnp
from jax import lax
from jax.experimental import pallas as pl
from jax.experimental.pallas import tpu as pltpu
from jax.experimental.pallas import tpu_sc as plsc

ROW = 10000
MAIN = 9984
TAIL = ROW - MAIN
TPAD = 128
BATCH = 8192
SC_N = 4608
TC_N = BATCH - SC_N
NC, NS = 2, 16
NW = NC * NS
PER_W = SC_N // NW
K = 2
NGR = PER_W // K
LANES = 16
CHUNKS = MAIN // LANES
U = 24
NJ = CHUNKS // U
NACC = 6
TCHUNKS = TPAD // LANES
GPS = LANES // K
NSG = PER_W // LANES

BP = 128
NB = TC_N // BP


def _pair_dot(rm1, rt1, rm2, rt2, p):
    def body(j, accs):
        base = j * (U * LANES)
        accs = list(accs)
        for u in range(U):
            x = rm1[p, pl.ds(base + u * LANES, LANES)]
            y = rm2[p, pl.ds(base + u * LANES, LANES)]
            accs[u % NACC] = accs[u % NACC] + x * y
        return tuple(accs)

    accs = tuple(jnp.zeros((LANES,), jnp.float32) for _ in range(NACC))
    accs = lax.fori_loop(0, NJ, body, accs)
    accs = list(accs)
    for t in range(TCHUNKS):
        x = rt1[p, pl.ds(t * LANES, LANES)]
        y = rt2[p, pl.ds(t * LANES, LANES)]
        accs[t % NACC] = accs[t % NACC] + x * y
    tot = accs[0]
    for u in range(1, NACC):
        tot = tot + accs[u]
    lane = lax.broadcasted_iota(jnp.int32, (LANES,), 0)
    for sh in (8, 4, 2, 1):
        idx = jnp.bitwise_and(lane + sh, LANES - 1)
        tot = tot + tot.at[idx].get(mode="promise_in_bounds",
                                    unique_indices=True)
    return tot


def _sc_body(a_hbm, atail_hbm, n1_hbm, n2_hbm, out_hbm,
             idx1_v, idx2_v,
             m1a, m1b, m2a, m2b, t1a, t1b, t2a, t2b, out_v,
             s1a, s1b, s2a, s2b):
    wid = lax.axis_index("s") * NC + lax.axis_index("c")
    pltpu.sync_copy(n1_hbm.at[pl.ds(wid * NGR, NGR)], idx1_v)
    pltpu.sync_copy(n2_hbm.at[pl.ds(wid * NGR, NGR)], idx2_v)
    lane = lax.broadcasted_iota(jnp.int32, (LANES,), 0)

    a_main = a_hbm.at[:, pl.ds(0, MAIN)]
    m1 = (m1a, m1b)
    m2 = (m2a, m2b)
    t1 = (t1a, t1b)
    t2 = (t2a, t2b)
    sem1 = (s1a, s1b)
    sem2 = (s2a, s2b)

    def fire(g, par):
        pltpu.async_copy(a_main.at[idx1_v.at[g]], m1[par], sem1[par])
        pltpu.async_copy(atail_hbm.at[idx1_v.at[g]], t1[par], sem1[par])
        pltpu.async_copy(a_main.at[idx2_v.at[g]], m2[par], sem2[par])
        pltpu.async_copy(atail_hbm.at[idx2_v.at[g]], t2[par], sem2[par])

    def drain(par):
        pltpu.make_async_copy(a_main.at[idx1_v.at[0]], m1[par], sem1[par]).wait()
        pltpu.make_async_copy(atail_hbm.at[idx1_v.at[0]], t1[par], sem1[par]).wait()
        pltpu.make_async_copy(a_main.at[idx2_v.at[0]], m2[par], sem2[par]).wait()
        pltpu.make_async_copy(atail_hbm.at[idx2_v.at[0]], t2[par], sem2[par]).wait()

    fire(0, 0)

    def supergroup(sg, carry):
        vec = jnp.zeros((LANES,), jnp.float32)
        for q in range(GPS):
            par = q % 2
            g = sg * GPS + q
            drain(par)
            g_next = g + 1

            @pl.when(g_next < NGR)
            def _():
                fire(g_next, (q + 1) % 2)

            for p in range(K):
                s = _pair_dot(m1[par], t1[par], m2[par], t2[par], p)
                vec = jnp.where(lane == (q * K + p), s, vec)
        out_v[pl.ds(sg * LANES, LANES)] = vec
        return carry

    lax.fori_loop(0, NSG, supergroup, 0)
    pltpu.sync_copy(out_v, out_hbm.at[pl.ds(wid * PER_W, PER_W)])


def _tc_body(n1_sref, n2_sref, a_ref, out_ref, rows1, rows2, sems):
    i = pl.program_id(0)

    def fire(step, par):
        for p in range(BP):
            i1 = n1_sref[step * BP + p]
            i2 = n2_sref[step * BP + p]
            pltpu.make_async_copy(
                a_ref.at[pl.ds(i1, 1)], rows1.at[par, pl.ds(p, 1)],
                sems.at[par]).start()
            pltpu.make_async_copy(
                a_ref.at[pl.ds(i2, 1)], rows2.at[par, pl.ds(p, 1)],
                sems.at[par]).start()

    def drain(step, par):
        pltpu.make_async_copy(
            a_ref.at[pl.ds(0, BP)], rows1.at[par], sems.at[par]).wait()
        pltpu.make_async_copy(
            a_ref.at[pl.ds(0, BP)], rows2.at[par], sems.at[par]).wait()

    @pl.when(i == 0)
    def _():
        fire(0, 0)

    par = lax.rem(i, 2)

    @pl.when(i + 1 < NB)
    def _():
        fire(i + 1, lax.rem(i + 1, 2))

    drain(i, par)
    r1 = rows1[par]
    r2 = rows2[par]
    out_ref[0, 0, :] = jnp.sum(r1 * r2, axis=1)


def kernel(A, nodes1, nodes2):
    n1 = nodes1.astype(jnp.int32)
    n2 = nodes2.astype(jnp.int32)
    n1_sc = n1[:SC_N].reshape(SC_N // K, K)
    n2_sc = n2[:SC_N].reshape(SC_N // K, K)
    a_tail = jnp.pad(A[:, MAIN:], ((0, 0), (0, TPAD - TAIL)))
    mesh = plsc.VectorSubcoreMesh(core_axis_name="c", subcore_axis_name="s")
    sc_fn = pl.kernel(
        _sc_body,
        out_type=jax.ShapeDtypeStruct((SC_N,), jnp.float32),
        mesh=mesh,
        compiler_params=pltpu.CompilerParams(use_tc_tiling_on_sc=True),
        scratch_types=[
            pltpu.VMEM((NGR, K), jnp.int32),
            pltpu.VMEM((NGR, K), jnp.int32),
            pltpu.VMEM((K, MAIN), jnp.float32),
            pltpu.VMEM((K, MAIN), jnp.float32),
            pltpu.VMEM((K, MAIN), jnp.float32),
            pltpu.VMEM((K, MAIN), jnp.float32),
            pltpu.VMEM((K, TPAD), jnp.float32),
            pltpu.VMEM((K, TPAD), jnp.float32),
            pltpu.VMEM((K, TPAD), jnp.float32),
            pltpu.VMEM((K, TPAD), jnp.float32),
            pltpu.VMEM((PER_W,), jnp.float32),
            pltpu.SemaphoreType.DMA,
            pltpu.SemaphoreType.DMA,
            pltpu.SemaphoreType.DMA,
            pltpu.SemaphoreType.DMA,
        ],
    )
    sc_out = sc_fn(A, a_tail, n1_sc, n2_sc)

    tc_fn = pl.pallas_call(
        _tc_body,
        grid_spec=pltpu.PrefetchScalarGridSpec(
            num_scalar_prefetch=2,
            grid=(NB,),
            in_specs=[pl.BlockSpec(memory_space=pl.ANY)],
            out_specs=pl.BlockSpec((1, 1, BP), lambda i, n1, n2: (i, 0, 0)),
            scratch_shapes=[
                pltpu.VMEM((2, BP, ROW), jnp.float32),
                pltpu.VMEM((2, BP, ROW), jnp.float32),
                pltpu.SemaphoreType.DMA((2,)),
            ],
        ),
        out_shape=jax.ShapeDtypeStruct((NB, 1, BP), jnp.float32),
    )
    tc_out = tc_fn(n1[SC_N:], n2[SC_N:], A).reshape(TC_N)
    return jnp.concatenate([sc_out, tc_out])


# speedup vs baseline: 3.4686x; 1.0425x over previous
import jax
import jax.numpy as jnp
from jax import lax
from jax.experimental import pallas as pl
from jax.experimental.pallas import tpu as pltpu
from jax.experimental.pallas import tpu_sc as plsc

ROW = 10000
MAIN = 9984
TAIL = ROW - MAIN
TPAD = 128
BATCH = 8192
SC_N = 4096
TC_N = BATCH - SC_N
NC, NS = 2, 16
NW = NC * NS
PER_W = SC_N // NW
K = 2
NGR = PER_W // K
LANES = 16
CHUNKS = MAIN // LANES
U = 24
NJ = CHUNKS // U
NACC = 6
TCHUNKS = TPAD // LANES
GPS = LANES // K
NSG = PER_W // LANES

BP = 128
NB = TC_N // BP


def _pair_dot(rm1, rt1, rm2, rt2, p):
    def body(j, accs):
        base = j * (U * LANES)
        accs = list(accs)
        for u in range(U):
            x = rm1[p, pl.ds(base + u * LANES, LANES)]
            y = rm2[p, pl.ds(base + u * LANES, LANES)]
            accs[u % NACC] = accs[u % NACC] + x * y
        return tuple(accs)

    accs = tuple(jnp.zeros((LANES,), jnp.float32) for _ in range(NACC))
    accs = lax.fori_loop(0, NJ, body, accs)
    accs = list(accs)
    for t in range(TCHUNKS):
        x = rt1[p, pl.ds(t * LANES, LANES)]
        y = rt2[p, pl.ds(t * LANES, LANES)]
        accs[t % NACC] = accs[t % NACC] + x * y
    tot = accs[0]
    for u in range(1, NACC):
        tot = tot + accs[u]
    lane = lax.broadcasted_iota(jnp.int32, (LANES,), 0)
    for sh in (8, 4, 2, 1):
        idx = jnp.bitwise_and(lane + sh, LANES - 1)
        tot = tot + tot.at[idx].get(mode="promise_in_bounds",
                                    unique_indices=True)
    return tot


def _sc_body(a_hbm, atail_hbm, n1_hbm, n2_hbm, out_hbm,
             idx1_v, idx2_v,
             m1a, m1b, m2a, m2b, t1a, t1b, t2a, t2b, out_v,
             s1a, s1b, s2a, s2b):
    wid = lax.axis_index("s") * NC + lax.axis_index("c")
    pltpu.sync_copy(n1_hbm.at[pl.ds(wid * NGR, NGR)], idx1_v)
    pltpu.sync_copy(n2_hbm.at[pl.ds(wid * NGR, NGR)], idx2_v)
    lane = lax.broadcasted_iota(jnp.int32, (LANES,), 0)

    a_main = a_hbm.at[:, pl.ds(0, MAIN)]
    m1 = (m1a, m1b)
    m2 = (m2a, m2b)
    t1 = (t1a, t1b)
    t2 = (t2a, t2b)
    sem1 = (s1a, s1b)
    sem2 = (s2a, s2b)

    def fire(g, par):
        pltpu.async_copy(a_main.at[idx1_v.at[g]], m1[par], sem1[par])
        pltpu.async_copy(atail_hbm.at[idx1_v.at[g]], t1[par], sem1[par])
        pltpu.async_copy(a_main.at[idx2_v.at[g]], m2[par], sem2[par])
        pltpu.async_copy(atail_hbm.at[idx2_v.at[g]], t2[par], sem2[par])

    def drain(par):
        pltpu.make_async_copy(a_main.at[idx1_v.at[0]], m1[par], sem1[par]).wait()
        pltpu.make_async_copy(atail_hbm.at[idx1_v.at[0]], t1[par], sem1[par]).wait()
        pltpu.make_async_copy(a_main.at[idx2_v.at[0]], m2[par], sem2[par]).wait()
        pltpu.make_async_copy(atail_hbm.at[idx2_v.at[0]], t2[par], sem2[par]).wait()

    fire(0, 0)

    def supergroup(sg, carry):
        vec = jnp.zeros((LANES,), jnp.float32)
        for q in range(GPS):
            par = q % 2
            g = sg * GPS + q
            drain(par)
            g_next = g + 1

            @pl.when(g_next < NGR)
            def _():
                fire(g_next, (q + 1) % 2)

            for p in range(K):
                s = _pair_dot(m1[par], t1[par], m2[par], t2[par], p)
                vec = jnp.where(lane == (q * K + p), s, vec)
        out_v[pl.ds(sg * LANES, LANES)] = vec
        return carry

    lax.fori_loop(0, NSG, supergroup, 0)
    pltpu.sync_copy(out_v, out_hbm.at[pl.ds(wid * PER_W, PER_W)])


def _tc_body(n1_sref, n2_sref, a_ref, out_ref, rows1, rows2, sems):
    i = pl.program_id(0)

    def fire(step, par):
        for p in range(BP):
            i1 = n1_sref[step * BP + p]
            i2 = n2_sref[step * BP + p]
            pltpu.make_async_copy(
                a_ref.at[pl.ds(i1, 1)], rows1.at[par, pl.ds(p, 1)],
                sems.at[par]).start()
            pltpu.make_async_copy(
                a_ref.at[pl.ds(i2, 1)], rows2.at[par, pl.ds(p, 1)],
                sems.at[par]).start()

    def drain(step, par):
        pltpu.make_async_copy(
            a_ref.at[pl.ds(0, BP)], rows1.at[par], sems.at[par]).wait()
        pltpu.make_async_copy(
            a_ref.at[pl.ds(0, BP)], rows2.at[par], sems.at[par]).wait()

    @pl.when(i == 0)
    def _():
        fire(0, 0)

    par = lax.rem(i, 2)

    @pl.when(i + 1 < NB)
    def _():
        fire(i + 1, lax.rem(i + 1, 2))

    drain(i, par)
    r1 = rows1[par]
    r2 = rows2[par]
    out_ref[0, 0, :] = jnp.sum(r1 * r2, axis=1)


def kernel(A, nodes1, nodes2):
    n1 = nodes1.astype(jnp.int32)
    n2 = nodes2.astype(jnp.int32)
    n1_sc = n1[:SC_N].reshape(SC_N // K, K)
    n2_sc = n2[:SC_N].reshape(SC_N // K, K)
    a_tail = jnp.pad(A[:, MAIN:], ((0, 0), (0, TPAD - TAIL)))
    mesh = plsc.VectorSubcoreMesh(core_axis_name="c", subcore_axis_name="s")
    sc_fn = pl.kernel(
        _sc_body,
        out_type=jax.ShapeDtypeStruct((SC_N,), jnp.float32),
        mesh=mesh,
        compiler_params=pltpu.CompilerParams(use_tc_tiling_on_sc=True),
        scratch_types=[
            pltpu.VMEM((NGR, K), jnp.int32),
            pltpu.VMEM((NGR, K), jnp.int32),
            pltpu.VMEM((K, MAIN), jnp.float32),
            pltpu.VMEM((K, MAIN), jnp.float32),
            pltpu.VMEM((K, MAIN), jnp.float32),
            pltpu.VMEM((K, MAIN), jnp.float32),
            pltpu.VMEM((K, TPAD), jnp.float32),
            pltpu.VMEM((K, TPAD), jnp.float32),
            pltpu.VMEM((K, TPAD), jnp.float32),
            pltpu.VMEM((K, TPAD), jnp.float32),
            pltpu.VMEM((PER_W,), jnp.float32),
            pltpu.SemaphoreType.DMA,
            pltpu.SemaphoreType.DMA,
            pltpu.SemaphoreType.DMA,
            pltpu.SemaphoreType.DMA,
        ],
    )
    sc_out = sc_fn(A, a_tail, n1_sc, n2_sc)

    tc_fn = pl.pallas_call(
        _tc_body,
        grid_spec=pltpu.PrefetchScalarGridSpec(
            num_scalar_prefetch=2,
            grid=(NB,),
            in_specs=[pl.BlockSpec(memory_space=pl.ANY)],
            out_specs=pl.BlockSpec((1, 1, BP), lambda i, n1, n2: (i, 0, 0)),
            scratch_shapes=[
                pltpu.VMEM((2, BP, ROW), jnp.float32),
                pltpu.VMEM((2, BP, ROW), jnp.float32),
                pltpu.SemaphoreType.DMA((2,)),
            ],
        ),
        out_shape=jax.ShapeDtypeStruct((NB, 1, BP), jnp.float32),
    )
    tc_out = tc_fn(n1[SC_N:], n2[SC_N:], A).reshape(TC_N)
    return jnp.concatenate([sc_out, tc_out])


# speedup vs baseline: 3.5912x; 1.0353x over previous
import jax
import jax.numpy as jnp
from jax import lax
from jax.experimental import pallas as pl
from jax.experimental.pallas import tpu as pltpu
from jax.experimental.pallas import tpu_sc as plsc

ROW = 10000
MAIN = 9984
TAIL = ROW - MAIN
TPAD = 128
BATCH = 8192
SC_N = 3584
TC_N = BATCH - SC_N
NC, NS = 2, 16
NW = NC * NS
PER_W = SC_N // NW
K = 2
NGR = PER_W // K
LANES = 16
CHUNKS = MAIN // LANES
U = 24
NJ = CHUNKS // U
NACC = 6
TCHUNKS = TPAD // LANES
GPS = LANES // K
NSG = PER_W // LANES

BP = 128
NB = TC_N // BP


def _pair_dot(rm1, rt1, rm2, rt2, p):
    def body(j, accs):
        base = j * (U * LANES)
        accs = list(accs)
        for u in range(U):
            x = rm1[p, pl.ds(base + u * LANES, LANES)]
            y = rm2[p, pl.ds(base + u * LANES, LANES)]
            accs[u % NACC] = accs[u % NACC] + x * y
        return tuple(accs)

    accs = tuple(jnp.zeros((LANES,), jnp.float32) for _ in range(NACC))
    accs = lax.fori_loop(0, NJ, body, accs)
    accs = list(accs)
    for t in range(TCHUNKS):
        x = rt1[p, pl.ds(t * LANES, LANES)]
        y = rt2[p, pl.ds(t * LANES, LANES)]
        accs[t % NACC] = accs[t % NACC] + x * y
    tot = accs[0]
    for u in range(1, NACC):
        tot = tot + accs[u]
    lane = lax.broadcasted_iota(jnp.int32, (LANES,), 0)
    for sh in (8, 4, 2, 1):
        idx = jnp.bitwise_and(lane + sh, LANES - 1)
        tot = tot + tot.at[idx].get(mode="promise_in_bounds",
                                    unique_indices=True)
    return tot


def _sc_body(a_hbm, atail_hbm, n1_hbm, n2_hbm, out_hbm,
             idx1_v, idx2_v,
             m1a, m1b, m2a, m2b, t1a, t1b, t2a, t2b, out_v,
             s1a, s1b, s2a, s2b):
    wid = lax.axis_index("s") * NC + lax.axis_index("c")
    pltpu.sync_copy(n1_hbm.at[pl.ds(wid * NGR, NGR)], idx1_v)
    pltpu.sync_copy(n2_hbm.at[pl.ds(wid * NGR, NGR)], idx2_v)
    lane = lax.broadcasted_iota(jnp.int32, (LANES,), 0)

    a_main = a_hbm.at[:, pl.ds(0, MAIN)]
    m1 = (m1a, m1b)
    m2 = (m2a, m2b)
    t1 = (t1a, t1b)
    t2 = (t2a, t2b)
    sem1 = (s1a, s1b)
    sem2 = (s2a, s2b)

    def fire(g, par):
        pltpu.async_copy(a_main.at[idx1_v.at[g]], m1[par], sem1[par])
        pltpu.async_copy(atail_hbm.at[idx1_v.at[g]], t1[par], sem1[par])
        pltpu.async_copy(a_main.at[idx2_v.at[g]], m2[par], sem2[par])
        pltpu.async_copy(atail_hbm.at[idx2_v.at[g]], t2[par], sem2[par])

    def drain(par):
        pltpu.make_async_copy(a_main.at[idx1_v.at[0]], m1[par], sem1[par]).wait()
        pltpu.make_async_copy(atail_hbm.at[idx1_v.at[0]], t1[par], sem1[par]).wait()
        pltpu.make_async_copy(a_main.at[idx2_v.at[0]], m2[par], sem2[par]).wait()
        pltpu.make_async_copy(atail_hbm.at[idx2_v.at[0]], t2[par], sem2[par]).wait()

    fire(0, 0)

    def supergroup(sg, carry):
        vec = jnp.zeros((LANES,), jnp.float32)
        for q in range(GPS):
            par = q % 2
            g = sg * GPS + q
            drain(par)
            g_next = g + 1

            @pl.when(g_next < NGR)
            def _():
                fire(g_next, (q + 1) % 2)

            for p in range(K):
                s = _pair_dot(m1[par], t1[par], m2[par], t2[par], p)
                vec = jnp.where(lane == (q * K + p), s, vec)
        out_v[pl.ds(sg * LANES, LANES)] = vec
        return carry

    lax.fori_loop(0, NSG, supergroup, 0)
    pltpu.sync_copy(out_v, out_hbm.at[pl.ds(wid * PER_W, PER_W)])


def _tc_body(n1_sref, n2_sref, a_ref, out_ref, rows1, rows2, sems):
    i = pl.program_id(0)

    def fire(step, par):
        for p in range(BP):
            i1 = n1_sref[step * BP + p]
            i2 = n2_sref[step * BP + p]
            pltpu.make_async_copy(
                a_ref.at[pl.ds(i1, 1)], rows1.at[par, pl.ds(p, 1)],
                sems.at[par]).start()
            pltpu.make_async_copy(
                a_ref.at[pl.ds(i2, 1)], rows2.at[par, pl.ds(p, 1)],
                sems.at[par]).start()

    def drain(step, par):
        pltpu.make_async_copy(
            a_ref.at[pl.ds(0, BP)], rows1.at[par], sems.at[par]).wait()
        pltpu.make_async_copy(
            a_ref.at[pl.ds(0, BP)], rows2.at[par], sems.at[par]).wait()

    @pl.when(i == 0)
    def _():
        fire(0, 0)

    par = lax.rem(i, 2)

    @pl.when(i + 1 < NB)
    def _():
        fire(i + 1, lax.rem(i + 1, 2))

    drain(i, par)
    r1 = rows1[par]
    r2 = rows2[par]
    out_ref[0, 0, :] = jnp.sum(r1 * r2, axis=1)


def kernel(A, nodes1, nodes2):
    n1 = nodes1.astype(jnp.int32)
    n2 = nodes2.astype(jnp.int32)
    n1_sc = n1[:SC_N].reshape(SC_N // K, K)
    n2_sc = n2[:SC_N].reshape(SC_N // K, K)
    a_tail = jnp.pad(A[:, MAIN:], ((0, 0), (0, TPAD - TAIL)))
    mesh = plsc.VectorSubcoreMesh(core_axis_name="c", subcore_axis_name="s")
    sc_fn = pl.kernel(
        _sc_body,
        out_type=jax.ShapeDtypeStruct((SC_N,), jnp.float32),
        mesh=mesh,
        compiler_params=pltpu.CompilerParams(use_tc_tiling_on_sc=True),
        scratch_types=[
            pltpu.VMEM((NGR, K), jnp.int32),
            pltpu.VMEM((NGR, K), jnp.int32),
            pltpu.VMEM((K, MAIN), jnp.float32),
            pltpu.VMEM((K, MAIN), jnp.float32),
            pltpu.VMEM((K, MAIN), jnp.float32),
            pltpu.VMEM((K, MAIN), jnp.float32),
            pltpu.VMEM((K, TPAD), jnp.float32),
            pltpu.VMEM((K, TPAD), jnp.float32),
            pltpu.VMEM((K, TPAD), jnp.float32),
            pltpu.VMEM((K, TPAD), jnp.float32),
            pltpu.VMEM((PER_W,), jnp.float32),
            pltpu.SemaphoreType.DMA,
            pltpu.SemaphoreType.DMA,
            pltpu.SemaphoreType.DMA,
            pltpu.SemaphoreType.DMA,
        ],
    )
    sc_out = sc_fn(A, a_tail, n1_sc, n2_sc)

    tc_fn = pl.pallas_call(
        _tc_body,
        grid_spec=pltpu.PrefetchScalarGridSpec(
            num_scalar_prefetch=2,
            grid=(NB,),
            in_specs=[pl.BlockSpec(memory_space=pl.ANY)],
            out_specs=pl.BlockSpec((1, 1, BP), lambda i, n1, n2: (i, 0, 0)),
            scratch_shapes=[
                pltpu.VMEM((2, BP, ROW), jnp.float32),
                pltpu.VMEM((2, BP, ROW), jnp.float32),
                pltpu.SemaphoreType.DMA((2,)),
            ],
        ),
        out_shape=jax.ShapeDtypeStruct((NB, 1, BP), jnp.float32),
    )
    tc_out = tc_fn(n1[SC_N:], n2[SC_N:], A).reshape(TC_N)
    return jnp.concatenate([sc_out, tc_out])
